# Initial kernel scaffold; baseline (speedup 1.0000x reference)
import jax
import jax.numpy as jnp
from jax.experimental import pallas as pl


def kernel(features, mol_edge_index, ddi_edge_index, W1, b1, Wp1, bp1, W2, b2, Wp2, bp2, W3, b3, Wp3, bp3, Wd, bd, Wc, bc):
    raise NotImplementedError("write your pallas kernel here")


# speedup vs baseline: 8.9898x; 8.9898x over previous
import functools

import jax
import jax.numpy as jnp
from jax import lax
from jax.experimental import pallas as pl
from jax.experimental.pallas import tpu as pltpu
from jax.experimental.pallas import tpu_sc as plsc

M = 2048
N = 64
E = 128
DDE = 65536
NHID = 128
DDI_NHID = 256
NUM_LABELS = 200
K1, K2, K3 = 32, 16, 8

BM = 8
SC_CORES = 2
SC_SUBCORES = 16
SC_WORKERS = SC_CORES * SC_SUBCORES
EPW = DDE // SC_WORKERS
ECH = 128
NCH = EPW // ECH
ROWS_PER_SUB = M // SC_SUBCORES

_F32 = jnp.float32


def _mm(a, b):
    return lax.dot_general(a, b, (((1,), (0,)), ((), ())),
                           preferred_element_type=_F32)


def _mmT(a, b):
    return lax.dot_general(a, b, (((1,), (1,)), ((), ())),
                           preferred_element_type=_F32)


def _tcol(v, ident):
    return lax.dot_general(v, ident, (((0,), (0,)), ((), ())),
                           preferred_element_type=_F32)


def _eye(n):
    r = lax.broadcasted_iota(jnp.int32, (n, n), 0)
    c = lax.broadcasted_iota(jnp.int32, (n, n), 1)
    return jnp.where(r == c, 1.0, 0.0).astype(_F32)


def _norm_adj(b_mat, n):
    ident = _eye(n)
    deg = jnp.sum(b_mat, axis=1, keepdims=True) + 1.0
    dinv = 1.0 / jnp.sqrt(deg)
    dinv_row = _tcol(dinv, ident)
    return (b_mat + ident) * dinv * dinv_row, ident


def _pool(x, s_col, b_mat, ident, k, n):
    s_row = _tcol(s_col, ident)
    gt = s_row > s_col
    eq = s_row == s_col
    ii = lax.broadcasted_iota(jnp.int32, (n, n), 0)
    jj = lax.broadcasted_iota(jnp.int32, (n, n), 1)
    before = jnp.where(gt | (eq & (jj < ii)), 1.0, 0.0)
    rank_col = jnp.sum(before, axis=1, keepdims=True)
    rank_row = _tcol(rank_col, ident)
    sel = lax.broadcasted_iota(jnp.int32, (k, n), 0)
    rank_i = rank_row.astype(jnp.int32)
    p_mat = jnp.where(sel == rank_i, 1.0, 0.0).astype(_F32)
    topv = _mm(p_mat, s_col)
    xk = _mm(p_mat, x) * jnp.tanh(topv)
    b_new = _mm(p_mat, _mmT(b_mat, p_mat))
    return xk, b_new


def _mol_body(feat_ref, ei_ref, w1_ref, b1_ref, wp1_ref, bp1_ref,
              w2_ref, b2_ref, wp2_ref, bp2_ref,
              w3_ref, b3_ref, wp3_ref, bp3_ref, out_ref):
    w1 = w1_ref[...]
    b1 = b1_ref[...]
    wp1 = wp1_ref[...][:, 0:1]
    bp1 = bp1_ref[...][:, 0:1]
    w2 = w2_ref[...]
    b2 = b2_ref[...]
    wp2 = wp2_ref[...][:, 0:1]
    bp2 = bp2_ref[...][:, 0:1]
    w3 = w3_ref[...]
    b3 = b3_ref[...]
    wp3 = wp3_ref[...][:, 0:1]
    bp3 = bp3_ref[...][:, 0:1]

    feats = jnp.reshape(feat_ref[...], (BM * N, 3))
    h1_all = _mm(feats, w1)

    for g in range(BM):
        edges = ei_ref[g]
        src = edges[0:1, :]
        dst = edges[1:2, :]
        rowid = lax.broadcasted_iota(jnp.int32, (N, E), 0)
        d_hot = jnp.where(rowid == dst, 1.0, 0.0).astype(_F32)
        s_hot = jnp.where(rowid == src, 1.0, 0.0).astype(_F32)
        b_mat = _mmT(d_hot, s_hot)

        a_mat, ident = _norm_adj(b_mat, N)
        x = jax.nn.relu(_mm(a_mat, h1_all[g * N:(g + 1) * N, :]) + b1)
        s_col = _mm(a_mat, _mm(x, wp1)) + bp1
        x, b_mat = _pool(x, s_col, b_mat, ident, K1, N)
        out_ref[pl.ds(g, 1), 0:NHID] = jnp.max(x, axis=0, keepdims=True)
        out_ref[pl.ds(g, 1), NHID:2 * NHID] = (
            jnp.sum(x, axis=0, keepdims=True) / K1)

        a_mat, ident = _norm_adj(b_mat, K1)
        x = jax.nn.relu(_mm(a_mat, _mm(x, w2)) + b2)
        s_col = _mm(a_mat, _mm(x, wp2)) + bp2
        x, b_mat = _pool(x, s_col, b_mat, ident, K2, K1)
        out_ref[pl.ds(g, 1), 2 * NHID:3 * NHID] = jnp.max(x, axis=0, keepdims=True)
        out_ref[pl.ds(g, 1), 3 * NHID:4 * NHID] = (
            jnp.sum(x, axis=0, keepdims=True) / K2)

        a_mat, ident = _norm_adj(b_mat, K2)
        x = jax.nn.relu(_mm(a_mat, _mm(x, w3)) + b3)
        s_col = _mm(a_mat, _mm(x, wp3)) + bp3
        x, b_mat = _pool(x, s_col, b_mat, ident, K3, K2)
        out_ref[pl.ds(g, 1), 4 * NHID:5 * NHID] = jnp.max(x, axis=0, keepdims=True)
        out_ref[pl.ds(g, 1), 5 * NHID:6 * NHID] = (
            jnp.sum(x, axis=0, keepdims=True) / K3)


def _full2d(shape):
    return pl.BlockSpec(shape, lambda i: (0, 0))


def _mol_call(features, mol_edge_index, w1, b1r, wp1, bp1r,
              w2, b2r, wp2, bp2r, w3, b3r, wp3, bp3r):
    return pl.pallas_call(
        _mol_body,
        grid=(M // BM,),
        in_specs=[
            pl.BlockSpec((BM, N, 3), lambda i: (i, 0, 0)),
            pl.BlockSpec((BM, 2, E), lambda i: (i, 0, 0)),
            _full2d((3, NHID)), _full2d((1, NHID)),
            _full2d((NHID, 1)), _full2d((1, NHID)),
            _full2d((NHID, NHID)), _full2d((1, NHID)),
            _full2d((NHID, 1)), _full2d((1, NHID)),
            _full2d((NHID, NHID)), _full2d((1, NHID)),
            _full2d((NHID, 1)), _full2d((1, NHID)),
        ],
        out_specs=pl.BlockSpec((BM, 6 * NHID), lambda i: (i, 0)),
        out_shape=jax.ShapeDtypeStruct((M, 6 * NHID), _F32),
    )(features, mol_edge_index, w1, b1r, wp1, bp1r,
      w2, b2r, wp2, bp2r, w3, b3r, wp3, bp3r)


def _deg_kernel_body(dst_hbm, out_hbm, idx_cur, hist):
    c = lax.axis_index("c")
    s = lax.axis_index("s")
    w = c * SC_SUBCORES + s
    one0 = jnp.where(lax.iota(jnp.int32, 16) == 0, 1.0, 0.0).astype(_F32)
    zeros = jnp.zeros((16,), _F32)

    def zbody(i, carry):
        hist[pl.ds(i * 16, 16)] = zeros
        return carry

    lax.fori_loop(0, (M + 16) // 16, zbody, 0)
    for j in range(NCH):
        pltpu.sync_copy(dst_hbm.at[w * NCH + j], idx_cur)

        def chunk(q, carry):
            dvec = idx_cur[pl.ds(q * 16, 16)]
            for l in range(16):
                d = dvec[l]
                hist[pl.ds(d, 16)] = hist[pl.ds(d, 16)] + one0
            return carry

        lax.fori_loop(0, ECH // 16, chunk, 0)
    pltpu.sync_copy(hist.at[pl.ds(0, M)], out_hbm.at[w])


def _deg_call(dst2d):
    mesh = plsc.VectorSubcoreMesh(core_axis_name="c", subcore_axis_name="s")
    fn = functools.partial(
        pl.kernel,
        mesh=mesh,
        out_type=jax.ShapeDtypeStruct((SC_WORKERS, M), _F32),
        scratch_types=[
            pltpu.VMEM((ECH,), jnp.int32),
            pltpu.VMEM((M + 16,), _F32),
        ],
    )(_deg_kernel_body)
    return fn(dst2d)


def _scat_kernel_body(g_hbm, src_hbm, dst_hbm, zeros_hbm, out_hbm,
                      src_cur, dst_cur, rows_v, sem, sem2):
    c = lax.axis_index("c")
    s = lax.axis_index("s")
    w = c * SC_SUBCORES + s
    pltpu.sync_copy(zeros_hbm, rows_v)
    for k in range(M // ECH):
        pltpu.sync_copy(rows_v, out_hbm.at[w, pl.ds(k * ECH, ECH)])
    for j in range(NCH):
        pltpu.sync_copy(src_hbm.at[w * NCH + j], src_cur)
        pltpu.sync_copy(dst_hbm.at[w * NCH + j], dst_cur)
        pltpu.async_copy(g_hbm.at[src_cur], rows_v, sem).wait()
        pltpu.async_copy(rows_v, out_hbm.at[w].at[dst_cur], sem2,
                         add=True).wait()


def _scat_call(g_rows, src2d, dst2d, zeros256):
    mesh = plsc.VectorSubcoreMesh(core_axis_name="c", subcore_axis_name="s")
    fn = functools.partial(
        pl.kernel,
        mesh=mesh,
        out_type=jax.ShapeDtypeStruct((SC_WORKERS, M, DDI_NHID), _F32),
        scratch_types=[
            pltpu.VMEM((ECH,), jnp.int32),
            pltpu.VMEM((ECH,), jnp.int32),
            pltpu.VMEM((ECH, DDI_NHID), _F32),
            pltpu.SemaphoreType.DMA,
            pltpu.SemaphoreType.DMA,
        ],
    )(_scat_kernel_body)
    return fn(g_rows, src2d, dst2d, zeros256)


def _dinv_diag(degp, n):
    row = degp[0:1, :]
    for k in range(1, SC_WORKERS):
        row = row + degp[k:k + 1, :]
    dinv_row = 1.0 / jnp.sqrt(row + 1.0)
    return _eye(n) * dinv_row


def _ddi_h_body(mol_ref, wd_ref, degp_ref, h_ref, g_ref):
    diag = _dinv_diag(degp_ref[...], mol_ref.shape[0])
    h = _mm(mol_ref[...], wd_ref[...])
    h_ref[...] = h
    g_ref[...] = _mm(diag, h)


def _ddi_h_call(mol, wd, deg_parts):
    bm = 256
    return pl.pallas_call(
        _ddi_h_body,
        grid=(M // bm,),
        in_specs=[
            pl.BlockSpec((bm, 6 * NHID), lambda i: (i, 0)),
            pl.BlockSpec((6 * NHID, DDI_NHID), lambda i: (0, 0)),
            pl.BlockSpec((SC_WORKERS, bm), lambda i: (0, i)),
        ],
        out_specs=[
            pl.BlockSpec((bm, DDI_NHID), lambda i: (i, 0)),
            pl.BlockSpec((bm, DDI_NHID), lambda i: (i, 0)),
        ],
        out_shape=[
            jax.ShapeDtypeStruct((M, DDI_NHID), _F32),
            jax.ShapeDtypeStruct((M, DDI_NHID), _F32),
        ],
    )(mol, wd, deg_parts)


def _ddi_out_body(acc_ref, h_ref, degp_ref, bd_ref, wc_ref, bc_ref, out_ref):
    diag = _dinv_diag(degp_ref[...], h_ref.shape[0])
    acc = acc_ref[...]
    agg = acc[0]
    for k in range(1, SC_WORKERS):
        agg = agg + acc[k]
    h = h_ref[...]
    u = jax.nn.relu(_mm(diag, agg + _mm(diag, h)) + bd_ref[...])
    logits = _mm(u, wc_ref[...]) + bc_ref[...]
    mx = jnp.max(logits, axis=1, keepdims=True)
    sh = logits - mx
    out_ref[...] = sh - jnp.log(jnp.sum(jnp.exp(sh), axis=1, keepdims=True))


def _ddi_out_call(acc_parts, h, deg_parts, bdr, wc, bcr):
    bm = 256
    return pl.pallas_call(
        _ddi_out_body,
        grid=(M // bm,),
        in_specs=[
            pl.BlockSpec((SC_WORKERS, bm, DDI_NHID), lambda i: (0, i, 0)),
            pl.BlockSpec((bm, DDI_NHID), lambda i: (i, 0)),
            pl.BlockSpec((SC_WORKERS, bm), lambda i: (0, i)),
            _full2d((1, DDI_NHID)),
            _full2d((DDI_NHID, NUM_LABELS)),
            _full2d((1, NUM_LABELS)),
        ],
        out_specs=pl.BlockSpec((bm, NUM_LABELS), lambda i: (i, 0)),
        out_shape=jax.ShapeDtypeStruct((M, NUM_LABELS), _F32),
    )(acc_parts, h, deg_parts, bdr, wc, bcr)


def kernel(features, mol_edge_index, ddi_edge_index, W1, b1, Wp1, bp1,
           W2, b2, Wp2, bp2, W3, b3, Wp3, bp3, Wd, bd, Wc, bc):
    row = lambda v, n: jnp.broadcast_to(jnp.reshape(v, (1, -1)), (1, n))
    b1r = jnp.reshape(b1, (1, NHID))
    b2r = jnp.reshape(b2, (1, NHID))
    b3r = jnp.reshape(b3, (1, NHID))
    bp1r = row(bp1, NHID)
    bp2r = row(bp2, NHID)
    bp3r = row(bp3, NHID)
    bdr = jnp.reshape(bd, (1, DDI_NHID))
    bcr = jnp.reshape(bc, (1, NUM_LABELS))

    src2d = jnp.reshape(ddi_edge_index[0], (DDE // ECH, ECH))
    dst2d = jnp.reshape(ddi_edge_index[1], (DDE // ECH, ECH))
    zeros256 = jnp.zeros((ECH, DDI_NHID), _F32)

    deg_parts = _deg_call(dst2d)
    mol = _mol_call(features, mol_edge_index, W1, b1r, Wp1, bp1r,
                    W2, b2r, Wp2, bp2r, W3, b3r, Wp3, bp3r)
    h, g_rows = _ddi_h_call(mol, Wd, deg_parts)
    acc_parts = _scat_call(g_rows, src2d, dst2d, zeros256)
    return _ddi_out_call(acc_parts, h, deg_parts, bdr, Wc, bcr)


# speedup vs baseline: 43.0802x; 4.7921x over previous
import functools

import jax
import jax.numpy as jnp
from jax import lax
from jax.experimental import pallas as pl
from jax.experimental.pallas import tpu as pltpu
from jax.experimental.pallas import tpu_sc as plsc

M = 2048
N = 64
E = 128
DDE = 65536
NHID = 128
DDI_NHID = 256
NUM_LABELS = 200
K1, K2, K3 = 32, 16, 8

BM = 8
PG = 8
SC_CORES = 2
SC_SUBCORES = 16
SC_WORKERS = SC_CORES * SC_SUBCORES
EPW = DDE // SC_WORKERS
ECH = 128
NCH = EPW // ECH
ROWS_PER_SUB = M // SC_SUBCORES

_F32 = jnp.float32


def _mm(a, b):
    return lax.dot_general(a, b, (((1,), (0,)), ((), ())),
                           preferred_element_type=_F32)


def _mmT(a, b):
    return lax.dot_general(a, b, (((1,), (1,)), ((), ())),
                           preferred_element_type=_F32)


def _mmH(a, b):
    return _mm(a, b)


def _tcol(v, ident):
    return lax.dot_general(v, ident, (((0,), (0,)), ((), ())),
                           preferred_element_type=_F32)


def _eye(n):
    r = lax.broadcasted_iota(jnp.int32, (n, n), 0)
    c = lax.broadcasted_iota(jnp.int32, (n, n), 1)
    return jnp.where(r == c, 1.0, 0.0).astype(_F32)


def _norm_adj(b_mat, n):
    ident = _eye(n)
    deg = jnp.sum(b_mat, axis=1, keepdims=True) + 1.0
    dinv = 1.0 / jnp.sqrt(deg)
    dinv_row = _tcol(dinv, ident)
    return (b_mat + ident) * dinv * dinv_row, ident


def _pool_blk(x, s_col, b_mat, ident, n, gs, kg):
    nk = (n // gs) * kg
    s_row = _tcol(s_col, ident)
    gt = s_row > s_col
    eq = s_row == s_col
    ii = lax.broadcasted_iota(jnp.int32, (n, n), 0)
    jj = lax.broadcasted_iota(jnp.int32, (n, n), 1)
    same = (ii // gs) == (jj // gs)
    before = jnp.where(same & (gt | (eq & (jj < ii))), 1.0, 0.0)
    rank_col = jnp.sum(before, axis=1, keepdims=True)
    rank_row = _tcol(rank_col, ident).astype(jnp.int32)
    oo = lax.broadcasted_iota(jnp.int32, (nk, n), 0)
    cc = lax.broadcasted_iota(jnp.int32, (nk, n), 1)
    p_mat = jnp.where(((cc // gs) == (oo // kg)) & (rank_row == (oo % kg)),
                      1.0, 0.0).astype(_F32)
    topv = _mmH(p_mat, s_col)
    xk = _mmH(p_mat, x) * jnp.tanh(topv)
    b_new = _mm(p_mat, _mmT(b_mat, p_mat))
    return xk, b_new


def _readout(x, kg, out_ref, row, g, off):
    xg = x[g * kg:(g + 1) * kg, :]
    out_ref[pl.ds(row, 1), pl.ds(off, NHID)] = jnp.max(xg, axis=0, keepdims=True)
    out_ref[pl.ds(row, 1), pl.ds(off + NHID, NHID)] = (
        jnp.sum(xg, axis=0, keepdims=True) / kg)


def _mol_body(feat_ref, ei_ref, w1_ref, b1_ref, wp1_ref, bp1_ref,
              w2_ref, b2_ref, wp2_ref, bp2_ref,
              w3_ref, b3_ref, wp3_ref, bp3_ref, out_ref, bblk0_ref, bblk1_ref):
    w1 = w1_ref[...]
    b1 = b1_ref[...]
    wp1 = wp1_ref[...][:, 0:1]
    bp1 = bp1_ref[...][:, 0:1]
    w2 = w2_ref[...]
    b2 = b2_ref[...]
    wp2 = wp2_ref[...][:, 0:1]
    bp2 = bp2_ref[...][:, 0:1]
    w3 = w3_ref[...]
    b3 = b3_ref[...]
    wp3 = wp3_ref[...][:, 0:1]
    bp3 = bp3_ref[...][:, 0:1]

    n1 = PG * N
    feats_all = jnp.reshape(feat_ref[...], (BM * N, 3))

    for pk in range(BM // PG):
        g0 = pk * PG
        feats = feats_all[pk * n1:(pk + 1) * n1, :]
        bblk_ref = bblk0_ref if pk == 0 else bblk1_ref

        bblk_ref[...] = jnp.zeros((n1, n1), _F32)
        for g in range(PG):
            edges = ei_ref[g0 + g]
            src = edges[0:1, :]
            dst = edges[1:2, :]
            rowid = lax.broadcasted_iota(jnp.int32, (N, E), 0)
            d_hot = jnp.where(rowid == dst, 1.0, 0.0).astype(jnp.bfloat16)
            s_hot = jnp.where(rowid == src, 1.0, 0.0).astype(jnp.bfloat16)
            bblk_ref[g * N:(g + 1) * N, g * N:(g + 1) * N] = (
                _mmT(d_hot, s_hot))

        b_mat = bblk_ref[...]

        a_mat, ident = _norm_adj(b_mat, n1)
        x = jax.nn.relu(_mmH(a_mat, _mm(feats, w1)) + b1)
        s_col = _mmH(a_mat, _mm(x, wp1)) + bp1
        x, b_mat = _pool_blk(x, s_col, b_mat, ident, n1, N, K1)
        for g in range(PG):
            _readout(x, K1, out_ref, g0 + g, g, 0)

        n2 = PG * K1
        a_mat, ident = _norm_adj(b_mat, n2)
        x = jax.nn.relu(_mmH(a_mat, _mm(x, w2)) + b2)
        s_col = _mmH(a_mat, _mm(x, wp2)) + bp2
        x, b_mat = _pool_blk(x, s_col, b_mat, ident, n2, K1, K2)
        for g in range(PG):
            _readout(x, K2, out_ref, g0 + g, g, 2 * NHID)

        n3 = PG * K2
        a_mat, ident = _norm_adj(b_mat, n3)
        x = jax.nn.relu(_mmH(a_mat, _mm(x, w3)) + b3)
        s_col = _mmH(a_mat, _mm(x, wp3)) + bp3
        x, b_mat = _pool_blk(x, s_col, b_mat, ident, n3, K2, K3)
        for g in range(PG):
            _readout(x, K3, out_ref, g0 + g, g, 4 * NHID)


def _full2d(shape):
    return pl.BlockSpec(shape, lambda i: (0, 0))


def _mol_call(features, mol_edge_index, w1, b1r, wp1, bp1r,
              w2, b2r, wp2, bp2r, w3, b3r, wp3, bp3r):
    return pl.pallas_call(
        _mol_body,
        grid=(M // BM,),
        in_specs=[
            pl.BlockSpec((BM, N, 3), lambda i: (i, 0, 0)),
            pl.BlockSpec((BM, 2, E), lambda i: (i, 0, 0)),
            _full2d((3, NHID)), _full2d((1, NHID)),
            _full2d((NHID, 1)), _full2d((1, NHID)),
            _full2d((NHID, NHID)), _full2d((1, NHID)),
            _full2d((NHID, 1)), _full2d((1, NHID)),
            _full2d((NHID, NHID)), _full2d((1, NHID)),
            _full2d((NHID, 1)), _full2d((1, NHID)),
        ],
        out_specs=pl.BlockSpec((BM, 6 * NHID), lambda i: (i, 0)),
        out_shape=jax.ShapeDtypeStruct((M, 6 * NHID), _F32),
        scratch_shapes=[pltpu.VMEM((PG * N, PG * N), _F32),
                        pltpu.VMEM((PG * N, PG * N), _F32)],
    )(features, mol_edge_index, w1, b1r, wp1, bp1r,
      w2, b2r, wp2, bp2r, w3, b3r, wp3, bp3r)


def _deg_kernel_body(dst_hbm, out_hbm, idx_cur, hist):
    c = lax.axis_index("c")
    s = lax.axis_index("s")
    w = c * SC_SUBCORES + s
    one0 = jnp.where(lax.iota(jnp.int32, 16) == 0, 1.0, 0.0).astype(_F32)
    zeros = jnp.zeros((16,), _F32)

    def zbody(i, carry):
        hist[pl.ds(i * 16, 16)] = zeros
        return carry

    lax.fori_loop(0, (M + 16) // 16, zbody, 0)
    for j in range(NCH):
        pltpu.sync_copy(dst_hbm.at[w * NCH + j], idx_cur)

        def chunk(q, carry):
            dvec = idx_cur[pl.ds(q * 16, 16)]
            for l in range(16):
                d = dvec[l]
                hist[pl.ds(d, 16)] = hist[pl.ds(d, 16)] + one0
            return carry

        lax.fori_loop(0, ECH // 16, chunk, 0)
    pltpu.sync_copy(hist.at[pl.ds(0, M)], out_hbm.at[w])


def _deg_call(dst2d):
    mesh = plsc.VectorSubcoreMesh(core_axis_name="c", subcore_axis_name="s")
    fn = functools.partial(
        pl.kernel,
        mesh=mesh,
        out_type=jax.ShapeDtypeStruct((SC_WORKERS, M), _F32),
        scratch_types=[
            pltpu.VMEM((ECH,), jnp.int32),
            pltpu.VMEM((M + 16,), _F32),
        ],
    )(_deg_kernel_body)
    return fn(dst2d)


def _scat_kernel_body(g_hbm, src_hbm, dst_hbm, zeros_hbm, out_hbm,
                      src_cur, dst_cur, rows_v, sem, sem2):
    c = lax.axis_index("c")
    s = lax.axis_index("s")
    w = c * SC_SUBCORES + s
    pltpu.sync_copy(zeros_hbm, rows_v)
    for k in range(M // ECH):
        pltpu.sync_copy(rows_v, out_hbm.at[w, pl.ds(k * ECH, ECH)])
    for j in range(NCH):
        pltpu.sync_copy(src_hbm.at[w * NCH + j], src_cur)
        pltpu.sync_copy(dst_hbm.at[w * NCH + j], dst_cur)
        pltpu.async_copy(g_hbm.at[src_cur], rows_v, sem).wait()
        pltpu.async_copy(rows_v, out_hbm.at[w].at[dst_cur], sem2,
                         add=True).wait()


def _scat_call(g_rows, src2d, dst2d, zeros256):
    mesh = plsc.VectorSubcoreMesh(core_axis_name="c", subcore_axis_name="s")
    fn = functools.partial(
        pl.kernel,
        mesh=mesh,
        out_type=jax.ShapeDtypeStruct((SC_WORKERS, M, DDI_NHID), _F32),
        scratch_types=[
            pltpu.VMEM((ECH,), jnp.int32),
            pltpu.VMEM((ECH,), jnp.int32),
            pltpu.VMEM((ECH, DDI_NHID), _F32),
            pltpu.SemaphoreType.DMA,
            pltpu.SemaphoreType.DMA,
        ],
    )(_scat_kernel_body)
    return fn(g_rows, src2d, dst2d, zeros256)


def _dinv_diag(degp, n):
    row = degp[0:1, :]
    for k in range(1, SC_WORKERS):
        row = row + degp[k:k + 1, :]
    dinv_row = 1.0 / jnp.sqrt(row + 1.0)
    return _eye(n) * dinv_row


def _ddi_h_body(mol_ref, wd_ref, degp_ref, h_ref, g_ref):
    diag = _dinv_diag(degp_ref[...], mol_ref.shape[0])
    h = _mm(mol_ref[...], wd_ref[...])
    h_ref[...] = h
    g_ref[...] = _mm(diag, h)


def _ddi_h_call(mol, wd, deg_parts):
    bm = 256
    return pl.pallas_call(
        _ddi_h_body,
        grid=(M // bm,),
        in_specs=[
            pl.BlockSpec((bm, 6 * NHID), lambda i: (i, 0)),
            pl.BlockSpec((6 * NHID, DDI_NHID), lambda i: (0, 0)),
            pl.BlockSpec((SC_WORKERS, bm), lambda i: (0, i)),
        ],
        out_specs=[
            pl.BlockSpec((bm, DDI_NHID), lambda i: (i, 0)),
            pl.BlockSpec((bm, DDI_NHID), lambda i: (i, 0)),
        ],
        out_shape=[
            jax.ShapeDtypeStruct((M, DDI_NHID), _F32),
            jax.ShapeDtypeStruct((M, DDI_NHID), _F32),
        ],
    )(mol, wd, deg_parts)


def _ddi_out_body(acc_ref, h_ref, degp_ref, bd_ref, wc_ref, bc_ref, out_ref):
    diag = _dinv_diag(degp_ref[...], h_ref.shape[0])
    acc = acc_ref[...]
    agg = acc[0]
    for k in range(1, SC_WORKERS):
        agg = agg + acc[k]
    h = h_ref[...]
    u = jax.nn.relu(_mm(diag, agg + _mm(diag, h)) + bd_ref[...])
    logits = _mm(u, wc_ref[...]) + bc_ref[...]
    mx = jnp.max(logits, axis=1, keepdims=True)
    sh = logits - mx
    out_ref[...] = sh - jnp.log(jnp.sum(jnp.exp(sh), axis=1, keepdims=True))


def _ddi_out_call(acc_parts, h, deg_parts, bdr, wc, bcr):
    bm = 256
    return pl.pallas_call(
        _ddi_out_body,
        grid=(M // bm,),
        in_specs=[
            pl.BlockSpec((SC_WORKERS, bm, DDI_NHID), lambda i: (0, i, 0)),
            pl.BlockSpec((bm, DDI_NHID), lambda i: (i, 0)),
            pl.BlockSpec((SC_WORKERS, bm), lambda i: (0, i)),
            _full2d((1, DDI_NHID)),
            _full2d((DDI_NHID, NUM_LABELS)),
            _full2d((1, NUM_LABELS)),
        ],
        out_specs=pl.BlockSpec((bm, NUM_LABELS), lambda i: (i, 0)),
        out_shape=jax.ShapeDtypeStruct((M, NUM_LABELS), _F32),
    )(acc_parts, h, deg_parts, bdr, wc, bcr)


def kernel(features, mol_edge_index, ddi_edge_index, W1, b1, Wp1, bp1,
           W2, b2, Wp2, bp2, W3, b3, Wp3, bp3, Wd, bd, Wc, bc):
    row = lambda v, n: jnp.broadcast_to(jnp.reshape(v, (1, -1)), (1, n))
    b1r = jnp.reshape(b1, (1, NHID))
    b2r = jnp.reshape(b2, (1, NHID))
    b3r = jnp.reshape(b3, (1, NHID))
    bp1r = row(bp1, NHID)
    bp2r = row(bp2, NHID)
    bp3r = row(bp3, NHID)
    bdr = jnp.reshape(bd, (1, DDI_NHID))
    bcr = jnp.reshape(bc, (1, NUM_LABELS))

    src2d = jnp.reshape(ddi_edge_index[0], (DDE // ECH, ECH))
    dst2d = jnp.reshape(ddi_edge_index[1], (DDE // ECH, ECH))
    zeros256 = jnp.zeros((ECH, DDI_NHID), _F32)

    deg_parts = _deg_call(dst2d)
    mol = _mol_call(features, mol_edge_index, W1, b1r, Wp1, bp1r,
                    W2, b2r, Wp2, bp2r, W3, b3r, Wp3, bp3r)
    h, g_rows = _ddi_h_call(mol, Wd, deg_parts)
    acc_parts = _scat_call(g_rows, src2d, dst2d, zeros256)
    return _ddi_out_call(acc_parts, h, deg_parts, bdr, Wc, bcr)


# speedup vs baseline: 63.7404x; 1.4796x over previous
import functools

import jax
import jax.numpy as jnp
from jax import lax
from jax.experimental import pallas as pl
from jax.experimental.pallas import tpu as pltpu
from jax.experimental.pallas import tpu_sc as plsc

M = 2048
N = 64
E = 128
DDE = 65536
NHID = 128
DDI_NHID = 256
NUM_LABELS = 200
K1, K2, K3 = 32, 16, 8

BM = 32
PG = 8
P2 = 4
SC_CORES = 2
SC_SUBCORES = 16
SC_WORKERS = SC_CORES * SC_SUBCORES
EPW = DDE // SC_WORKERS
ECH = 128
NCH = EPW // ECH
ROWS_PER_SUB = M // SC_SUBCORES

_F32 = jnp.float32


def _mm(a, b):
    return lax.dot_general(a, b, (((1,), (0,)), ((), ())),
                           preferred_element_type=_F32)


def _mmT(a, b):
    return lax.dot_general(a, b, (((1,), (1,)), ((), ())),
                           preferred_element_type=_F32)


def _mmH(a, b):
    return _mm(a, b)


def _tcol(v, ident):
    return lax.dot_general(v, ident, (((0,), (0,)), ((), ())),
                           preferred_element_type=_F32)


def _eye(n):
    r = lax.broadcasted_iota(jnp.int32, (n, n), 0)
    c = lax.broadcasted_iota(jnp.int32, (n, n), 1)
    return jnp.where(r == c, 1.0, 0.0).astype(_F32)


def _norm_adj(b_mat, n):
    ident = _eye(n)
    deg = jnp.sum(b_mat, axis=1, keepdims=True) + 1.0
    dinv = 1.0 / jnp.sqrt(deg)
    dinv_row = _tcol(dinv, ident)
    return (b_mat + ident) * dinv * dinv_row, ident


def _pool_blk(x, s_col, b_mat, ident, n, gs, kg):
    nk = (n // gs) * kg
    s_row = _tcol(s_col, ident)
    gt = s_row > s_col
    eq = s_row == s_col
    ii = lax.broadcasted_iota(jnp.int32, (n, n), 0)
    jj = lax.broadcasted_iota(jnp.int32, (n, n), 1)
    same = (ii // gs) == (jj // gs)
    before = jnp.where(same & (gt | (eq & (jj < ii))), 1.0, 0.0)
    rank_col = jnp.sum(before, axis=1, keepdims=True)
    rank_row = _tcol(rank_col, ident).astype(jnp.int32)
    oo = lax.broadcasted_iota(jnp.int32, (nk, n), 0)
    cc = lax.broadcasted_iota(jnp.int32, (nk, n), 1)
    p_mat = jnp.where(((cc // gs) == (oo // kg)) & (rank_row == (oo % kg)),
                      1.0, 0.0).astype(_F32)
    topv = _mmH(p_mat, s_col)
    xk = _mmH(p_mat, x) * jnp.tanh(topv)
    b_new = _mm(p_mat, _mmT(b_mat, p_mat))
    return xk, b_new


def _readout(x, kg, out_ref, row, g, off):
    xg = x[g * kg:(g + 1) * kg, :]
    out_ref[pl.ds(row, 1), pl.ds(off, NHID)] = jnp.max(xg, axis=0, keepdims=True)
    out_ref[pl.ds(row, 1), pl.ds(off + NHID, NHID)] = (
        jnp.sum(xg, axis=0, keepdims=True) / kg)


def _mol_body(feat_ref, ei_ref, w1_ref, b1_ref, wp1_ref, bp1_ref,
              w2_ref, b2_ref, wp2_ref, bp2_ref,
              w3_ref, b3_ref, wp3_ref, bp3_ref, out_ref,
              bblk0_ref, bblk1_ref, bblk2_ref, bblk3_ref):
    w1 = w1_ref[...]
    b1 = b1_ref[...]
    wp1 = wp1_ref[...][:, 0:1]
    bp1 = bp1_ref[...][:, 0:1]
    w2 = w2_ref[...]
    b2 = b2_ref[...]
    wp2 = wp2_ref[...][:, 0:1]
    bp2 = bp2_ref[...][:, 0:1]
    w3 = w3_ref[...]
    b3 = b3_ref[...]
    wp3 = wp3_ref[...][:, 0:1]
    bp3 = bp3_ref[...][:, 0:1]

    n1 = PG * N
    n2 = PG * K1
    n3 = PG * K2
    feats_all = jnp.reshape(feat_ref[...], (BM * N, 3))
    scr = [bblk0_ref, bblk1_ref, bblk2_ref, bblk3_ref]

    for pk in range(P2):
        scr[pk][...] = jnp.zeros((n1, n1), _F32)
    for g in range(P2 * PG):
        edges = ei_ref[g]
        src = edges[0:1, :]
        dst = edges[1:2, :]
        rowid = lax.broadcasted_iota(jnp.int32, (N, E), 0)
        d_hot = jnp.where(rowid == dst, 1.0, 0.0).astype(jnp.bfloat16)
        s_hot = jnp.where(rowid == src, 1.0, 0.0).astype(jnp.bfloat16)
        lg = g % PG
        scr[g // PG][lg * N:(lg + 1) * N, lg * N:(lg + 1) * N] = (
            _mmT(d_hot, s_hot))

    feats_l = [feats_all[pk * n1 * 0 + pk * PG * N:(pk + 1) * PG * N, :]
               for pk in range(P2)]
    b_l = [scr[pk][...] for pk in range(P2)]

    def layer(b_l, x_l, w, bvec, wp, bpv, n, gs, kg, first_feats=None):
        ai = [_norm_adj(b, n) for b in b_l]
        if first_feats is not None:
            x_l = first_feats
        t_l = [_mm(x, w) for x in x_l]
        x_l = [jax.nn.relu(_mmH(a, tt) + bvec) for (a, _), tt in zip(ai, t_l)]
        s_l = [_mmH(a, _mm(x, wp)) + bpv for (a, _), x in zip(ai, x_l)]
        pooled = [_pool_blk(x, s, b, ident, n, gs, kg)
                  for x, s, b, (a, ident) in zip(x_l, s_l, b_l, ai)]
        x_l = [pb[0] for pb in pooled]
        b_l = [pb[1] for pb in pooled]
        return x_l, b_l

    x_l, b_l = layer(b_l, None, w1, b1, wp1, bp1, n1, N, K1,
                     first_feats=feats_l)
    for pk in range(P2):
        for g in range(PG):
            _readout(x_l[pk], K1, out_ref, pk * PG + g, g, 0)

    x_l, b_l = layer(b_l, x_l, w2, b2, wp2, bp2, n2, K1, K2)
    for pk in range(P2):
        for g in range(PG):
            _readout(x_l[pk], K2, out_ref, pk * PG + g, g, 2 * NHID)

    x_l, b_l = layer(b_l, x_l, w3, b3, wp3, bp3, n3, K2, K3)
    for pk in range(P2):
        for g in range(PG):
            _readout(x_l[pk], K3, out_ref, pk * PG + g, g, 4 * NHID)


def _full2d(shape):
    return pl.BlockSpec(shape, lambda i: (0, 0))


def _mol_call(features, mol_edge_index, w1, b1r, wp1, bp1r,
              w2, b2r, wp2, bp2r, w3, b3r, wp3, bp3r):
    return pl.pallas_call(
        _mol_body,
        grid=(M // BM,),
        in_specs=[
            pl.BlockSpec((BM, N, 3), lambda i: (i, 0, 0)),
            pl.BlockSpec((BM, 2, E), lambda i: (i, 0, 0)),
            _full2d((3, NHID)), _full2d((1, NHID)),
            _full2d((NHID, 1)), _full2d((1, NHID)),
            _full2d((NHID, NHID)), _full2d((1, NHID)),
            _full2d((NHID, 1)), _full2d((1, NHID)),
            _full2d((NHID, NHID)), _full2d((1, NHID)),
            _full2d((NHID, 1)), _full2d((1, NHID)),
        ],
        out_specs=pl.BlockSpec((BM, 6 * NHID), lambda i: (i, 0)),
        out_shape=jax.ShapeDtypeStruct((M, 6 * NHID), _F32),
        scratch_shapes=[pltpu.VMEM((PG * N, PG * N), _F32),
                        pltpu.VMEM((PG * N, PG * N), _F32),
                        pltpu.VMEM((PG * N, PG * N), _F32),
                        pltpu.VMEM((PG * N, PG * N), _F32)],
    )(features, mol_edge_index, w1, b1r, wp1, bp1r,
      w2, b2r, wp2, bp2r, w3, b3r, wp3, bp3r)


def _deg_kernel_body(dst_hbm, out_hbm, idx_cur, hist):
    c = lax.axis_index("c")
    s = lax.axis_index("s")
    w = c * SC_SUBCORES + s
    one0 = jnp.where(lax.iota(jnp.int32, 16) == 0, 1.0, 0.0).astype(_F32)
    zeros = jnp.zeros((16,), _F32)

    def zbody(i, carry):
        hist[pl.ds(i * 16, 16)] = zeros
        return carry

    lax.fori_loop(0, (M + 16) // 16, zbody, 0)
    for j in range(NCH):
        pltpu.sync_copy(dst_hbm.at[w * NCH + j], idx_cur)

        def chunk(q, carry):
            dvec = idx_cur[pl.ds(q * 16, 16)]
            for l in range(16):
                d = dvec[l]
                hist[pl.ds(d, 16)] = hist[pl.ds(d, 16)] + one0
            return carry

        lax.fori_loop(0, ECH // 16, chunk, 0)
    pltpu.sync_copy(hist.at[pl.ds(0, M)], out_hbm.at[w])


def _deg_call(dst2d):
    mesh = plsc.VectorSubcoreMesh(core_axis_name="c", subcore_axis_name="s")
    fn = functools.partial(
        pl.kernel,
        mesh=mesh,
        out_type=jax.ShapeDtypeStruct((SC_WORKERS, M), _F32),
        scratch_types=[
            pltpu.VMEM((ECH,), jnp.int32),
            pltpu.VMEM((M + 16,), _F32),
        ],
    )(_deg_kernel_body)
    return fn(dst2d)


def _scat_kernel_body(g_hbm, src_hbm, dst_hbm, zeros_hbm, out_hbm,
                      src_cur, dst_cur, rows_v, sem, sem2):
    c = lax.axis_index("c")
    s = lax.axis_index("s")
    w = c * SC_SUBCORES + s
    pltpu.sync_copy(zeros_hbm, rows_v)
    for k in range(M // ECH):
        pltpu.sync_copy(rows_v, out_hbm.at[w, pl.ds(k * ECH, ECH)])
    for j in range(NCH):
        pltpu.sync_copy(src_hbm.at[w * NCH + j], src_cur)
        pltpu.sync_copy(dst_hbm.at[w * NCH + j], dst_cur)
        pltpu.async_copy(g_hbm.at[src_cur], rows_v, sem).wait()
        pltpu.async_copy(rows_v, out_hbm.at[w].at[dst_cur], sem2,
                         add=True).wait()


def _scat_call(g_rows, src2d, dst2d, zeros256):
    mesh = plsc.VectorSubcoreMesh(core_axis_name="c", subcore_axis_name="s")
    fn = functools.partial(
        pl.kernel,
        mesh=mesh,
        out_type=jax.ShapeDtypeStruct((SC_WORKERS, M, DDI_NHID), _F32),
        scratch_types=[
            pltpu.VMEM((ECH,), jnp.int32),
            pltpu.VMEM((ECH,), jnp.int32),
            pltpu.VMEM((ECH, DDI_NHID), _F32),
            pltpu.SemaphoreType.DMA,
            pltpu.SemaphoreType.DMA,
        ],
    )(_scat_kernel_body)
    return fn(g_rows, src2d, dst2d, zeros256)


def _dinv_diag(degp, n):
    row = degp[0:1, :]
    for k in range(1, SC_WORKERS):
        row = row + degp[k:k + 1, :]
    dinv_row = 1.0 / jnp.sqrt(row + 1.0)
    return _eye(n) * dinv_row


def _ddi_h_body(mol_ref, wd_ref, degp_ref, h_ref, g_ref):
    diag = _dinv_diag(degp_ref[...], mol_ref.shape[0])
    h = _mm(mol_ref[...], wd_ref[...])
    h_ref[...] = h
    g_ref[...] = _mm(diag, h)


def _ddi_h_call(mol, wd, deg_parts):
    bm = 256
    return pl.pallas_call(
        _ddi_h_body,
        grid=(M // bm,),
        in_specs=[
            pl.BlockSpec((bm, 6 * NHID), lambda i: (i, 0)),
            pl.BlockSpec((6 * NHID, DDI_NHID), lambda i: (0, 0)),
            pl.BlockSpec((SC_WORKERS, bm), lambda i: (0, i)),
        ],
        out_specs=[
            pl.BlockSpec((bm, DDI_NHID), lambda i: (i, 0)),
            pl.BlockSpec((bm, DDI_NHID), lambda i: (i, 0)),
        ],
        out_shape=[
            jax.ShapeDtypeStruct((M, DDI_NHID), _F32),
            jax.ShapeDtypeStruct((M, DDI_NHID), _F32),
        ],
    )(mol, wd, deg_parts)


def _ddi_out_body(acc_ref, h_ref, degp_ref, bd_ref, wc_ref, bc_ref, out_ref):
    diag = _dinv_diag(degp_ref[...], h_ref.shape[0])
    acc = acc_ref[...]
    agg = acc[0]
    for k in range(1, SC_WORKERS):
        agg = agg + acc[k]
    h = h_ref[...]
    u = jax.nn.relu(_mm(diag, agg + _mm(diag, h)) + bd_ref[...])
    logits = _mm(u, wc_ref[...]) + bc_ref[...]
    mx = jnp.max(logits, axis=1, keepdims=True)
    sh = logits - mx
    out_ref[...] = sh - jnp.log(jnp.sum(jnp.exp(sh), axis=1, keepdims=True))


def _ddi_out_call(acc_parts, h, deg_parts, bdr, wc, bcr):
    bm = 256
    return pl.pallas_call(
        _ddi_out_body,
        grid=(M // bm,),
        in_specs=[
            pl.BlockSpec((SC_WORKERS, bm, DDI_NHID), lambda i: (0, i, 0)),
            pl.BlockSpec((bm, DDI_NHID), lambda i: (i, 0)),
            pl.BlockSpec((SC_WORKERS, bm), lambda i: (0, i)),
            _full2d((1, DDI_NHID)),
            _full2d((DDI_NHID, NUM_LABELS)),
            _full2d((1, NUM_LABELS)),
        ],
        out_specs=pl.BlockSpec((bm, NUM_LABELS), lambda i: (i, 0)),
        out_shape=jax.ShapeDtypeStruct((M, NUM_LABELS), _F32),
    )(acc_parts, h, deg_parts, bdr, wc, bcr)


def kernel(features, mol_edge_index, ddi_edge_index, W1, b1, Wp1, bp1,
           W2, b2, Wp2, bp2, W3, b3, Wp3, bp3, Wd, bd, Wc, bc):
    row = lambda v, n: jnp.broadcast_to(jnp.reshape(v, (1, -1)), (1, n))
    b1r = jnp.reshape(b1, (1, NHID))
    b2r = jnp.reshape(b2, (1, NHID))
    b3r = jnp.reshape(b3, (1, NHID))
    bp1r = row(bp1, NHID)
    bp2r = row(bp2, NHID)
    bp3r = row(bp3, NHID)
    bdr = jnp.reshape(bd, (1, DDI_NHID))
    bcr = jnp.reshape(bc, (1, NUM_LABELS))

    src2d = jnp.reshape(ddi_edge_index[0], (DDE // ECH, ECH))
    dst2d = jnp.reshape(ddi_edge_index[1], (DDE // ECH, ECH))
    zeros256 = jnp.zeros((ECH, DDI_NHID), _F32)

    deg_parts = _deg_call(dst2d)
    mol = _mol_call(features, mol_edge_index, W1, b1r, Wp1, bp1r,
                    W2, b2r, Wp2, bp2r, W3, b3r, Wp3, bp3r)
    h, g_rows = _ddi_h_call(mol, Wd, deg_parts)
    acc_parts = _scat_call(g_rows, src2d, dst2d, zeros256)
    return _ddi_out_call(acc_parts, h, deg_parts, bdr, Wc, bcr)


# speedup vs baseline: 63.8665x; 1.0020x over previous
import functools

import jax
import jax.numpy as jnp
from jax import lax
from jax.experimental import pallas as pl
from jax.experimental.pallas import tpu as pltpu
from jax.experimental.pallas import tpu_sc as plsc

M = 2048
N = 64
E = 128
DDE = 65536
NHID = 128
DDI_NHID = 256
NUM_LABELS = 200
K1, K2, K3 = 32, 16, 8

BM = 32
PG = 8
P2 = 4
SC_CORES = 2
SC_SUBCORES = 16
SC_WORKERS = SC_CORES * SC_SUBCORES
EPW = DDE // SC_WORKERS
ECH = 128
NCH = EPW // ECH
ROWS_PER_SUB = M // SC_SUBCORES

_F32 = jnp.float32


def _mm(a, b):
    return lax.dot_general(a, b, (((1,), (0,)), ((), ())),
                           preferred_element_type=_F32)


def _mmT(a, b):
    return lax.dot_general(a, b, (((1,), (1,)), ((), ())),
                           preferred_element_type=_F32)


def _mmH(a, b):
    return _mm(a, b)


def _tcol(v, ident):
    return lax.dot_general(v, ident, (((0,), (0,)), ((), ())),
                           preferred_element_type=_F32)


def _eye(n):
    r = lax.broadcasted_iota(jnp.int32, (n, n), 0)
    c = lax.broadcasted_iota(jnp.int32, (n, n), 1)
    return jnp.where(r == c, 1.0, 0.0).astype(_F32)


def _norm_adj(b_mat, n):
    ident = _eye(n)
    deg = jnp.sum(b_mat, axis=1, keepdims=True) + 1.0
    dinv = 1.0 / jnp.sqrt(deg)
    dinv_row = _tcol(dinv, ident)
    return (b_mat + ident) * dinv * dinv_row, ident


def _pool_blk(x, s_col, b_mat, ident, n, gs, kg):
    nk = (n // gs) * kg
    s_row = _tcol(s_col, ident)
    gt = s_row > s_col
    eq = s_row == s_col
    ii = lax.broadcasted_iota(jnp.int32, (n, n), 0)
    jj = lax.broadcasted_iota(jnp.int32, (n, n), 1)
    same = (ii // gs) == (jj // gs)
    before = jnp.where(same & (gt | (eq & (jj < ii))), 1.0, 0.0)
    rank_col = jnp.sum(before, axis=1, keepdims=True)
    rank_row = _tcol(rank_col, ident).astype(jnp.int32)
    oo = lax.broadcasted_iota(jnp.int32, (nk, n), 0)
    cc = lax.broadcasted_iota(jnp.int32, (nk, n), 1)
    p_mat = jnp.where(((cc // gs) == (oo // kg)) & (rank_row == (oo % kg)),
                      1.0, 0.0).astype(_F32)
    topv = _mmH(p_mat, s_col)
    xk = _mmH(p_mat, x) * jnp.tanh(topv)
    b_new = _mm(p_mat, _mmT(b_mat, p_mat))
    return xk, b_new


def _readout(x, kg, out_ref, row, g, off):
    xg = x[g * kg:(g + 1) * kg, :]
    out_ref[pl.ds(row, 1), pl.ds(off, NHID)] = jnp.max(xg, axis=0, keepdims=True)
    out_ref[pl.ds(row, 1), pl.ds(off + NHID, NHID)] = (
        jnp.sum(xg, axis=0, keepdims=True) / kg)


def _mol_body(feat_ref, ei_ref, w1_ref, b1_ref, wp1_ref, bp1_ref,
              w2_ref, b2_ref, wp2_ref, bp2_ref,
              w3_ref, b3_ref, wp3_ref, bp3_ref, out_ref,
              bblk0_ref, bblk1_ref, bblk2_ref, bblk3_ref):
    w1 = w1_ref[...]
    b1 = b1_ref[...]
    wp1 = wp1_ref[...][:, 0:1]
    bp1 = bp1_ref[...][:, 0:1]
    w2 = w2_ref[...]
    b2 = b2_ref[...]
    wp2 = wp2_ref[...][:, 0:1]
    bp2 = bp2_ref[...][:, 0:1]
    w3 = w3_ref[...]
    b3 = b3_ref[...]
    wp3 = wp3_ref[...][:, 0:1]
    bp3 = bp3_ref[...][:, 0:1]

    n1 = PG * N
    n2 = PG * K1
    n3 = PG * K2
    feats_all = jnp.reshape(feat_ref[...], (BM * N, 3))
    scr = [bblk0_ref, bblk1_ref, bblk2_ref, bblk3_ref]

    for pk in range(P2):
        scr[pk][...] = jnp.zeros((n1, n1), _F32)
    for g in range(P2 * PG):
        edges = ei_ref[g]
        src = edges[0:1, :]
        dst = edges[1:2, :]
        rowid = lax.broadcasted_iota(jnp.int32, (N, E), 0)
        d_hot = jnp.where(rowid == dst, 1.0, 0.0).astype(jnp.bfloat16)
        s_hot = jnp.where(rowid == src, 1.0, 0.0).astype(jnp.bfloat16)
        lg = g % PG
        scr[g // PG][lg * N:(lg + 1) * N, lg * N:(lg + 1) * N] = (
            _mmT(d_hot, s_hot))

    feats_l = [feats_all[pk * n1 * 0 + pk * PG * N:(pk + 1) * PG * N, :]
               for pk in range(P2)]
    b_l = [scr[pk][...] for pk in range(P2)]

    def layer(b_l, x_l, w, bvec, wp, bpv, n, gs, kg, first_feats=None):
        ai = [_norm_adj(b, n) for b in b_l]
        if first_feats is not None:
            x_l = first_feats
        t_l = [_mm(x, w) for x in x_l]
        x_l = [jax.nn.relu(_mm(a, tt) + bvec) for (a, _), tt in zip(ai, t_l)]
        s_l = [_mm(a, _mm(x, wp)) + bpv for (a, _), x in zip(ai, x_l)]
        pooled = [_pool_blk(x, s, b, ident, n, gs, kg)
                  for x, s, b, (a, ident) in zip(x_l, s_l, b_l, ai)]
        x_l = [pb[0] for pb in pooled]
        b_l = [pb[1] for pb in pooled]
        return x_l, b_l

    x_l, b_l = layer(b_l, None, w1, b1, wp1, bp1, n1, N, K1,
                     first_feats=feats_l)
    for pk in range(P2):
        for g in range(PG):
            _readout(x_l[pk], K1, out_ref, pk * PG + g, g, 0)

    x_l, b_l = layer(b_l, x_l, w2, b2, wp2, bp2, n2, K1, K2)
    for pk in range(P2):
        for g in range(PG):
            _readout(x_l[pk], K2, out_ref, pk * PG + g, g, 2 * NHID)

    x_l, b_l = layer(b_l, x_l, w3, b3, wp3, bp3, n3, K2, K3)
    for pk in range(P2):
        for g in range(PG):
            _readout(x_l[pk], K3, out_ref, pk * PG + g, g, 4 * NHID)


def _full2d(shape):
    return pl.BlockSpec(shape, lambda i: (0, 0))


def _mol_call(features, mol_edge_index, w1, b1r, wp1, bp1r,
              w2, b2r, wp2, bp2r, w3, b3r, wp3, bp3r):
    return pl.pallas_call(
        _mol_body,
        grid=(M // BM,),
        in_specs=[
            pl.BlockSpec((BM, N, 3), lambda i: (i, 0, 0)),
            pl.BlockSpec((BM, 2, E), lambda i: (i, 0, 0)),
            _full2d((3, NHID)), _full2d((1, NHID)),
            _full2d((NHID, 1)), _full2d((1, NHID)),
            _full2d((NHID, NHID)), _full2d((1, NHID)),
            _full2d((NHID, 1)), _full2d((1, NHID)),
            _full2d((NHID, NHID)), _full2d((1, NHID)),
            _full2d((NHID, 1)), _full2d((1, NHID)),
        ],
        out_specs=pl.BlockSpec((BM, 6 * NHID), lambda i: (i, 0)),
        out_shape=jax.ShapeDtypeStruct((M, 6 * NHID), _F32),
        scratch_shapes=[pltpu.VMEM((PG * N, PG * N), _F32),
                        pltpu.VMEM((PG * N, PG * N), _F32),
                        pltpu.VMEM((PG * N, PG * N), _F32),
                        pltpu.VMEM((PG * N, PG * N), _F32)],
    )(features, mol_edge_index, w1, b1r, wp1, bp1r,
      w2, b2r, wp2, bp2r, w3, b3r, wp3, bp3r)


def _deg_kernel_body(dst_hbm, out_hbm, idx_cur, hist):
    c = lax.axis_index("c")
    s = lax.axis_index("s")
    w = c * SC_SUBCORES + s
    one0 = jnp.where(lax.iota(jnp.int32, 16) == 0, 1.0, 0.0).astype(_F32)
    zeros = jnp.zeros((16,), _F32)

    def zbody(i, carry):
        hist[pl.ds(i * 16, 16)] = zeros
        return carry

    lax.fori_loop(0, (M + 16) // 16, zbody, 0)
    for j in range(NCH):
        pltpu.sync_copy(dst_hbm.at[w * NCH + j], idx_cur)

        def chunk(q, carry):
            dvec = idx_cur[pl.ds(q * 16, 16)]
            for l in range(16):
                d = dvec[l]
                hist[pl.ds(d, 16)] = hist[pl.ds(d, 16)] + one0
            return carry

        lax.fori_loop(0, ECH // 16, chunk, 0)
    pltpu.sync_copy(hist.at[pl.ds(0, M)], out_hbm.at[w])


def _deg_call(dst2d):
    mesh = plsc.VectorSubcoreMesh(core_axis_name="c", subcore_axis_name="s")
    fn = functools.partial(
        pl.kernel,
        mesh=mesh,
        out_type=jax.ShapeDtypeStruct((SC_WORKERS, M), _F32),
        scratch_types=[
            pltpu.VMEM((ECH,), jnp.int32),
            pltpu.VMEM((M + 16,), _F32),
        ],
    )(_deg_kernel_body)
    return fn(dst2d)


def _scat_kernel_body(g_hbm, src_hbm, dst_hbm, zeros_hbm, out_hbm,
                      src_cur, dst_cur, rows_v, sem, sem2):
    c = lax.axis_index("c")
    s = lax.axis_index("s")
    w = c * SC_SUBCORES + s
    pltpu.sync_copy(zeros_hbm, rows_v)
    for k in range(M // ECH):
        pltpu.sync_copy(rows_v, out_hbm.at[w, pl.ds(k * ECH, ECH)])
    for j in range(NCH):
        pltpu.sync_copy(src_hbm.at[w * NCH + j], src_cur)
        pltpu.sync_copy(dst_hbm.at[w * NCH + j], dst_cur)
        pltpu.async_copy(g_hbm.at[src_cur], rows_v, sem).wait()
        pltpu.async_copy(rows_v, out_hbm.at[w].at[dst_cur], sem2,
                         add=True).wait()


def _scat_call(g_rows, src2d, dst2d, zeros256):
    mesh = plsc.VectorSubcoreMesh(core_axis_name="c", subcore_axis_name="s")
    fn = functools.partial(
        pl.kernel,
        mesh=mesh,
        out_type=jax.ShapeDtypeStruct((SC_WORKERS, M, DDI_NHID), _F32),
        scratch_types=[
            pltpu.VMEM((ECH,), jnp.int32),
            pltpu.VMEM((ECH,), jnp.int32),
            pltpu.VMEM((ECH, DDI_NHID), _F32),
            pltpu.SemaphoreType.DMA,
            pltpu.SemaphoreType.DMA,
        ],
    )(_scat_kernel_body)
    return fn(g_rows, src2d, dst2d, zeros256)


def _dinv_diag(degp, n):
    row = degp[0:1, :]
    for k in range(1, SC_WORKERS):
        row = row + degp[k:k + 1, :]
    dinv_row = 1.0 / jnp.sqrt(row + 1.0)
    return _eye(n) * dinv_row


def _ddi_h_body(mol_ref, wd_ref, degp_ref, h_ref, g_ref):
    diag = _dinv_diag(degp_ref[...], mol_ref.shape[0])
    h = _mm(mol_ref[...], wd_ref[...])
    h_ref[...] = h
    g_ref[...] = _mm(diag, h)


def _ddi_h_call(mol, wd, deg_parts):
    bm = 256
    return pl.pallas_call(
        _ddi_h_body,
        grid=(M // bm,),
        in_specs=[
            pl.BlockSpec((bm, 6 * NHID), lambda i: (i, 0)),
            pl.BlockSpec((6 * NHID, DDI_NHID), lambda i: (0, 0)),
            pl.BlockSpec((SC_WORKERS, bm), lambda i: (0, i)),
        ],
        out_specs=[
            pl.BlockSpec((bm, DDI_NHID), lambda i: (i, 0)),
            pl.BlockSpec((bm, DDI_NHID), lambda i: (i, 0)),
        ],
        out_shape=[
            jax.ShapeDtypeStruct((M, DDI_NHID), _F32),
            jax.ShapeDtypeStruct((M, DDI_NHID), _F32),
        ],
    )(mol, wd, deg_parts)


def _ddi_out_body(acc_ref, h_ref, degp_ref, bd_ref, wc_ref, bc_ref, out_ref):
    diag = _dinv_diag(degp_ref[...], h_ref.shape[0])
    acc = acc_ref[...]
    agg = acc[0]
    for k in range(1, SC_WORKERS):
        agg = agg + acc[k]
    h = h_ref[...]
    u = jax.nn.relu(_mm(diag, agg + _mm(diag, h)) + bd_ref[...])
    logits = _mm(u, wc_ref[...]) + bc_ref[...]
    mx = jnp.max(logits, axis=1, keepdims=True)
    sh = logits - mx
    out_ref[...] = sh - jnp.log(jnp.sum(jnp.exp(sh), axis=1, keepdims=True))


def _ddi_out_call(acc_parts, h, deg_parts, bdr, wc, bcr):
    bm = 256
    return pl.pallas_call(
        _ddi_out_body,
        grid=(M // bm,),
        in_specs=[
            pl.BlockSpec((SC_WORKERS, bm, DDI_NHID), lambda i: (0, i, 0)),
            pl.BlockSpec((bm, DDI_NHID), lambda i: (i, 0)),
            pl.BlockSpec((SC_WORKERS, bm), lambda i: (0, i)),
            _full2d((1, DDI_NHID)),
            _full2d((DDI_NHID, NUM_LABELS)),
            _full2d((1, NUM_LABELS)),
        ],
        out_specs=pl.BlockSpec((bm, NUM_LABELS), lambda i: (i, 0)),
        out_shape=jax.ShapeDtypeStruct((M, NUM_LABELS), _F32),
    )(acc_parts, h, deg_parts, bdr, wc, bcr)


def kernel(features, mol_edge_index, ddi_edge_index, W1, b1, Wp1, bp1,
           W2, b2, Wp2, bp2, W3, b3, Wp3, bp3, Wd, bd, Wc, bc):
    row = lambda v, n: jnp.broadcast_to(jnp.reshape(v, (1, -1)), (1, n))
    b1r = jnp.reshape(b1, (1, NHID))
    b2r = jnp.reshape(b2, (1, NHID))
    b3r = jnp.reshape(b3, (1, NHID))
    bp1r = row(bp1, NHID)
    bp2r = row(bp2, NHID)
    bp3r = row(bp3, NHID)
    bdr = jnp.reshape(bd, (1, DDI_NHID))
    bcr = jnp.reshape(bc, (1, NUM_LABELS))

    src2d = jnp.reshape(ddi_edge_index[0], (DDE // ECH, ECH))
    dst2d = jnp.reshape(ddi_edge_index[1], (DDE // ECH, ECH))
    zeros256 = jnp.zeros((ECH, DDI_NHID), _F32)

    deg_parts = _deg_call(dst2d)
    mol = _mol_call(features, mol_edge_index, W1, b1r, Wp1, bp1r,
                    W2, b2r, Wp2, bp2r, W3, b3r, Wp3, bp3r)
    h, g_rows = _ddi_h_call(mol, Wd, deg_parts)
    acc_parts = _scat_call(g_rows, src2d, dst2d, zeros256)
    return _ddi_out_call(acc_parts, h, deg_parts, bdr, Wc, bcr)
